# baseline (speedup 1.0000x reference)
import functools

import jax
import jax.numpy as jnp
from jax import lax
from jax.experimental import pallas as pl
from jax.experimental.pallas import tpu as pltpu
from jax.experimental.pallas import tpu_sc as plsc

N = 10000
C = 128
NC = 2
NS = 16
ZR = 16
RPT = 624
TAIL = N - NS * RPT
K = 40
B = 5
L = 2


def _mm_pm_body(x_ref, w_ref, op_ref, on_ref):
    a = jnp.dot(x_ref[...], w_ref[...], preferred_element_type=jnp.float32)
    op_ref[...] = a
    on_ref[...] = -a


def _sig_mm_pm_body(x_ref, w_ref, op_ref, on_ref):
    s = jax.nn.sigmoid(x_ref[...])
    a = jnp.dot(s, w_ref[...], preferred_element_type=jnp.float32)
    op_ref[...] = a
    on_ref[...] = -a


def _mm_sig_body(x_ref, w_ref, o_ref):
    o_ref[...] = jax.nn.sigmoid(
        jnp.dot(x_ref[...], w_ref[...], preferred_element_type=jnp.float32))


def _mm_add_sig_body(x_ref, w_ref, b_ref, o_ref):
    o_ref[...] = jax.nn.sigmoid(
        jnp.dot(x_ref[...], w_ref[...], preferred_element_type=jnp.float32)
        + b_ref[...])


def _row_spec(blk):
    return pl.BlockSpec((blk, C), lambda i: (i, 0))


def _w_spec():
    return pl.BlockSpec((C, C), lambda i: (0, 0))


def _tc_pm(body, x, w, blk):
    rows = x.shape[0]
    return pl.pallas_call(
        body,
        grid=(rows // blk,),
        in_specs=[_row_spec(blk), _w_spec()],
        out_specs=[_row_spec(blk), _row_spec(blk)],
        out_shape=[jax.ShapeDtypeStruct((rows, C), jnp.float32)] * 2,
    )(x, w)


def _tc_mm_sig(x, w, blk):
    rows = x.shape[0]
    return pl.pallas_call(
        _mm_sig_body,
        grid=(rows // blk,),
        in_specs=[_row_spec(blk), _w_spec()],
        out_specs=_row_spec(blk),
        out_shape=jax.ShapeDtypeStruct((rows, C), jnp.float32),
    )(x, w)


def _tc_mm_add_sig(x, w, b, blk):
    rows = x.shape[0]
    return pl.pallas_call(
        _mm_add_sig_body,
        grid=(rows // blk,),
        in_specs=[_row_spec(blk), _w_spec(), _row_spec(blk)],
        out_specs=_row_spec(blk),
        out_shape=jax.ShapeDtypeStruct((rows, C), jnp.float32),
    )(x, w, b)


def _zero_acc_slice(acc, zbuf, tid, sem):
    def zb(i, _):
        zbuf[i // (C // 16), pl.ds((i % (C // 16)) * 16, 16)] = (
            jnp.zeros((16,), jnp.float32))
        return 0

    lax.fori_loop(0, ZR * (C // 16), zb, 0)

    def zcopy(i, _):
        pltpu.async_copy(zbuf, acc.at[pl.ds(tid * RPT + i * ZR, ZR)], sem)
        return 0

    lax.fori_loop(0, RPT // ZR, zcopy, 0)

    def zdrain(i, _):
        pltpu.make_async_copy(zbuf, acc.at[pl.ds(tid * RPT + i * ZR, ZR)],
                              sem).wait()
        return 0

    lax.fori_loop(0, RPT // ZR, zdrain, 0)

    @pl.when(tid == NS - 1)
    def _():
        pltpu.sync_copy(zbuf.at[pl.ds(0, TAIL)], acc.at[pl.ds(NS * RPT, TAIL)])


def _acc_writeback(acc, out_hbm, tid):
    pltpu.sync_copy(acc.at[pl.ds(tid * RPT, RPT)],
                    out_hbm.at[pl.ds(tid * RPT, RPT)])

    @pl.when(tid == NS - 1)
    def _():
        pltpu.sync_copy(acc.at[pl.ds(NS * RPT, TAIL)],
                        out_hbm.at[pl.ds(NS * RPT, TAIL)])


def _ring_scatter(read_src_fn, didx_hbm, acc, rows, didx, semg, sems, nch,
                  base_t):
    def issue_read(i, c):
        pltpu.async_copy(read_src_fn(i), rows[c], semg[c])

    def wait_read(i, c):
        pltpu.make_async_copy(read_src_fn(i), rows[c], semg[c]).wait()

    for j in range(L):
        pltpu.sync_copy(didx_hbm.at[pl.ds(base_t + j * K, K)], didx[j])
        issue_read(j, j)

    def body(g, _):
        for b in range(B):
            i = g * B + b
            c = (b + L) % B
            wait_read(i, b)
            pltpu.async_copy(rows[b], acc.at[didx[b]], sems[b], add=True)

            @pl.when(i + L < nch)
            def _():
                @pl.when(i >= B - L)
                def _():
                    pltpu.make_async_copy(rows[c], acc.at[didx[c]],
                                          sems[c]).wait()

                pltpu.sync_copy(didx_hbm.at[pl.ds(base_t + (i + L) * K, K)],
                                didx[c])
                issue_read(i + L, c)

        return 0

    lax.fori_loop(0, nch // B, body, 0)
    for b in range(B):
        pltpu.make_async_copy(rows[b], acc.at[didx[b]], sems[b]).wait()


def _gather_diff_core(pos_hbm, neg_hbm, iv_hbm, iu_hbm, out_hbm, iall, rows,
                      semg, sema, semw, tid, ept):
    nch = ept // K
    base_t = tid * ept
    pltpu.sync_copy(iv_hbm.at[pl.ds(base_t, ept)], iall.at[pl.ds(0, ept)])
    pltpu.sync_copy(iu_hbm.at[pl.ds(base_t, ept)], iall.at[pl.ds(ept, ept)])

    def g1_src(i):
        return pos_hbm.at[iall.at[pl.ds(i * K, K)]]

    def g2_src(i):
        return neg_hbm.at[iall.at[pl.ds(ept + i * K, K)]]

    def out_dst(i):
        return out_hbm.at[pl.ds(base_t + i * K, K)]

    for j in range(L):
        pltpu.async_copy(g1_src(j), rows[j], semg[j])

    def body(g, _):
        for b in range(B):
            i = g * B + b
            bp = (b - 1) % B
            c = (b + L) % B

            @pl.when(i > 0)
            def _():
                pltpu.make_async_copy(g2_src(i - 1), rows[bp],
                                      sema[bp]).wait()
                pltpu.async_copy(rows[bp], out_dst(i - 1), semw[bp])

            pltpu.make_async_copy(g1_src(i), rows[b], semg[b]).wait()
            pltpu.async_copy(g2_src(i), rows[b], sema[b], add=True)

            @pl.when(i + L < nch)
            def _():
                @pl.when(i >= B - L)
                def _():
                    pltpu.make_async_copy(rows[c], out_dst(i + L - B),
                                          semw[c]).wait()

                pltpu.async_copy(g1_src(i + L), rows[c], semg[c])

        return 0

    lax.fori_loop(0, nch // B, body, 0)
    bl = (nch - 1) % B
    pltpu.make_async_copy(g2_src(nch - 1), rows[bl], sema[bl]).wait()
    pltpu.async_copy(rows[bl], out_dst(nch - 1), semw[bl])
    for b in range(B):
        pltpu.make_async_copy(rows[b], out_dst(0), semw[b]).wait()


def _sc_stage_a(x, xw2, xw2n, adj_src, adj_dst, inc_v, inc_u):
    EA = adj_src.shape[0]
    EI = inc_v.shape[0]
    ept_a = EA // NS
    ept_i = EI // NS
    mesh = plsc.VectorSubcoreMesh(core_axis_name="c", subcore_axis_name="s")

    @functools.partial(
        pl.kernel,
        out_type=[jax.ShapeDtypeStruct((N, C), jnp.float32),
                  jax.ShapeDtypeStruct((EI, C), jnp.float32)],
        mesh=mesh,
        scratch_types=[
            pltpu.VMEM_SHARED((N, C), jnp.float32),
            pltpu.VMEM((2 * ept_i,), jnp.int32),
            [pltpu.VMEM((K, C), jnp.float32)] * B,
            [pltpu.VMEM((K,), jnp.int32)] * B,
            pltpu.VMEM((ZR, C), jnp.float32),
            [pltpu.SemaphoreType.DMA] * B,
            [pltpu.SemaphoreType.DMA] * B,
            [pltpu.SemaphoreType.DMA] * B,
            pltpu.SemaphoreType.DMA,
        ],
    )
    def k(x_hbm, xw2_hbm, xw2n_hbm, asrc_hbm, adst_hbm, iv_hbm, iu_hbm,
          gx_hbm, epre_hbm, acc, iall, rows, didx, zbuf, semg, sems, semw,
          semz):
        cid = lax.axis_index("c")
        tid = lax.axis_index("s")

        @pl.when(cid == 0)
        def _():
            nch = ept_a // K
            base_t = tid * ept_a
            _zero_acc_slice(acc, zbuf, tid, semz)
            pltpu.sync_copy(asrc_hbm.at[pl.ds(base_t, ept_a)],
                            iall.at[pl.ds(0, ept_a)])
            plsc.subcore_barrier()

            def read_src(i):
                return x_hbm.at[iall.at[pl.ds(i * K, K)]]

            _ring_scatter(read_src, adst_hbm, acc, rows, didx, semg, sems,
                          nch, base_t)
            plsc.subcore_barrier()
            _acc_writeback(acc, gx_hbm, tid)

        @pl.when(cid == 1)
        def _():
            _gather_diff_core(xw2_hbm, xw2n_hbm, iv_hbm, iu_hbm, epre_hbm,
                              iall, rows, semg, sems, semw, tid, ept_i)

    return k(x, xw2, xw2n, adj_src, adj_dst, inc_v, inc_u)


def _sc_stage_c(n1, ewp, ewn, adj_src, adj_dst, inc_v, inc_u):
    EA = adj_src.shape[0]
    EI = inc_v.shape[0]
    ept_a = EA // NS
    ept_i = EI // NS
    mesh = plsc.VectorSubcoreMesh(core_axis_name="c", subcore_axis_name="s")

    @functools.partial(
        pl.kernel,
        out_type=[jax.ShapeDtypeStruct((N, C), jnp.float32),
                  jax.ShapeDtypeStruct((N, C), jnp.float32)],
        mesh=mesh,
        scratch_types=[
            pltpu.VMEM_SHARED((N, C), jnp.float32),
            pltpu.VMEM((ept_a,), jnp.int32),
            [pltpu.VMEM((K, C), jnp.float32)] * B,
            [pltpu.VMEM((K,), jnp.int32)] * B,
            pltpu.VMEM((ZR, C), jnp.float32),
            [pltpu.SemaphoreType.DMA] * B,
            [pltpu.SemaphoreType.DMA] * B,
            pltpu.SemaphoreType.DMA,
        ],
    )
    def k(n1_hbm, ewp_hbm, ewn_hbm, asrc_hbm, adst_hbm, iv_hbm, iu_hbm,
          gn_hbm, e2_hbm, acc, iall, rows, didx, zbuf, semg, sems, semz):
        cid = lax.axis_index("c")
        tid = lax.axis_index("s")

        @pl.when(cid == 0)
        def _():
            nch = ept_a // K
            base_t = tid * ept_a
            _zero_acc_slice(acc, zbuf, tid, semz)
            pltpu.sync_copy(asrc_hbm.at[pl.ds(base_t, ept_a)],
                            iall.at[pl.ds(0, ept_a)])
            plsc.subcore_barrier()

            def read_src(i):
                return n1_hbm.at[iall.at[pl.ds(i * K, K)]]

            _ring_scatter(read_src, adst_hbm, acc, rows, didx, semg, sems,
                          nch, base_t)
            plsc.subcore_barrier()
            _acc_writeback(acc, gn_hbm, tid)

        @pl.when(cid == 1)
        def _():
            nch = ept_i // K
            base_t = tid * ept_i
            _zero_acc_slice(acc, zbuf, tid, semz)
            plsc.subcore_barrier()

            def read_p(i):
                return ewp_hbm.at[pl.ds(base_t + i * K, K)]

            def read_n(i):
                return ewn_hbm.at[pl.ds(base_t + i * K, K)]

            _ring_scatter(read_p, iv_hbm, acc, rows, didx, semg, sems, nch,
                          base_t)
            _ring_scatter(read_n, iu_hbm, acc, rows, didx, semg, sems, nch,
                          base_t)
            plsc.subcore_barrier()
            _acc_writeback(acc, e2_hbm, tid)

    return k(n1, ewp, ewn, adj_src, adj_dst, inc_v, inc_u)


def kernel(x, adj_src, adj_dst, inc_u, inc_v, W1, W2, W3, W4):
    xw2, xw2n = _tc_pm(_mm_pm_body, x, W2, blk=1000)
    gx, e_pre = _sc_stage_a(x, xw2, xw2n, adj_src, adj_dst, inc_v, inc_u)
    n1 = _tc_mm_sig(gx, W1, blk=1000)
    ewp, ewn = _tc_pm(_sig_mm_pm_body, e_pre, W4, blk=2000)
    gn, e2 = _sc_stage_c(n1, ewp, ewn, adj_src, adj_dst, inc_v, inc_u)
    return _tc_mm_add_sig(gn, W3, e2, blk=1000)
